# baseline (speedup 1.0000x reference)
import functools

import jax
import jax.numpy as jnp
from jax import lax
from jax.experimental import pallas as pl
from jax.experimental.pallas import tpu as pltpu
from jax.experimental.pallas import tpu_sc as plsc

K = 8
EPS = 1e-8
N_S = 2048
N_F = 256
QT = 256

NC = 2
NS = 16
NW = NC * NS
QC = 16


def _tc_body(q_ref, sT_ref, i_ref, w_ref):
    b = pl.program_id(0)
    qq = q_ref[0]
    sT = sT_ref[...]
    qs = lax.dot_general(
        qq.astype(jnp.bfloat16),
        sT.astype(jnp.bfloat16),
        (((1,), (0,)), ((), ())),
        preferred_element_type=jnp.float32,
    )
    q2 = jnp.sum(qq * qq, axis=1, keepdims=True)
    s2 = jnp.sum(sT * sT, axis=0, keepdims=True)
    d2 = (q2 + s2) - 2.0 * qs
    d2 = jnp.maximum(d2, 0.0)
    iota = lax.broadcasted_iota(jnp.int32, (QT, N_S), 1)
    w_cols = []
    w_sum = jnp.zeros((QT, 1), jnp.float32)
    for k in range(K):
        m = jnp.min(d2, axis=1, keepdims=True)
        cand = jnp.where(d2 == m, iota, N_S)
        idx = jnp.min(cand, axis=1, keepdims=True)
        onehot = iota == idx
        w = 1.0 / (jnp.sqrt(m + 1e-12) + EPS)
        w_cols.append(w)
        w_sum = w_sum + w
        d2 = jnp.where(onehot, jnp.inf, d2)
        i_ref[0, :, k : k + 1] = idx + b * N_S
    for k in range(K):
        w_ref[0, :, k : k + 1] = w_cols[k] / w_sum


def _tc_topk(query_coords, s_t):
    B, n_q, _ = query_coords.shape
    return pl.pallas_call(
        _tc_body,
        grid=(B, n_q // QT),
        in_specs=[
            pl.BlockSpec((1, QT, 3), lambda b, t: (b, t, 0)),
            pl.BlockSpec((3, N_S), lambda b, t: (0, 0)),
        ],
        out_specs=(
            pl.BlockSpec((1, QT, K), lambda b, t: (b, t, 0)),
            pl.BlockSpec((1, QT, K), lambda b, t: (b, t, 0)),
        ),
        out_shape=(
            jax.ShapeDtypeStruct((B, n_q, K), jnp.int32),
            jax.ShapeDtypeStruct((B, n_q, K), jnp.float32),
        ),
    )(query_coords, s_t)


def _sc_combine(idx_flat, w_flat, feat_flat, n_total):
    q_per_w = n_total // NW
    n_chunks = q_per_w // QC
    mesh = plsc.VectorSubcoreMesh(core_axis_name="c", subcore_axis_name="s")

    @functools.partial(
        pl.kernel,
        mesh=mesh,
        out_type=jax.ShapeDtypeStruct((n_total, N_F), jnp.float32),
        scratch_types=[
            pltpu.VMEM((QC * K,), jnp.int32),
            pltpu.VMEM((QC * K,), jnp.float32),
            pltpu.VMEM((QC * K, N_F), jnp.float32),
            pltpu.VMEM((QC, N_F), jnp.float32),
            pltpu.SemaphoreType.DMA,
        ],
    )
    def body(idx_hbm, w_hbm, feat_hbm, out_hbm, idx_v, w_v, rows_v, out_v, sem):
        wid = lax.axis_index("s") * NC + lax.axis_index("c")
        qbase0 = wid * q_per_w

        def chunk(c, carry):
            qbase = qbase0 + c * QC
            pltpu.sync_copy(idx_hbm.at[pl.ds(qbase * K, QC * K)], idx_v)
            pltpu.sync_copy(w_hbm.at[pl.ds(qbase * K, QC * K)], w_v)
            pltpu.async_copy(feat_hbm.at[idx_v], rows_v, sem).wait()

            def one_pair(qp, carry2):
                wv = w_v[pl.ds(qp * 16, 16)]
                for h in range(2):
                    ql = qp * 2 + h
                    base = ql * K
                    accs = [jnp.zeros((16,), jnp.float32) for _ in range(N_F // 16)]
                    for k in range(K):
                        wk = wv[h * K + k]
                        for j in range(N_F // 16):
                            accs[j] = accs[j] + wk * rows_v[base + k, pl.ds(j * 16, 16)]
                    for j in range(N_F // 16):
                        out_v[ql, pl.ds(j * 16, 16)] = accs[j]
                return carry2

            lax.fori_loop(0, QC // 2, one_pair, 0)
            pltpu.sync_copy(out_v, out_hbm.at[pl.ds(qbase, QC)])
            return carry

        lax.fori_loop(0, n_chunks, chunk, 0)

    return body(idx_flat, w_flat, feat_flat)


@jax.jit
def kernel(query_coords, sensor_coords, sensor_features):
    B, n_q, _ = query_coords.shape
    s_t = sensor_coords.T
    idx, w = _tc_topk(query_coords, s_t)
    n_total = B * n_q
    out = _sc_combine(
        idx.reshape(n_total * K),
        w.reshape(n_total * K),
        sensor_features.reshape(B * N_S, N_F),
        n_total,
    )
    return out.reshape(B, n_q, N_F)


# speedup vs baseline: 1.2017x; 1.2017x over previous
import functools

import jax
import jax.numpy as jnp
from jax import lax
from jax.experimental import pallas as pl
from jax.experimental.pallas import tpu as pltpu
from jax.experimental.pallas import tpu_sc as plsc

K = 8
EPS = 1e-8
N_S = 2048
N_F = 256
QT = 512

NC = 2
NS = 16
NW = NC * NS
QC = 16


def _tc_body(q_ref, sT_ref, i_ref, w_ref):
    b = pl.program_id(0)
    qq = q_ref[0]
    sT = sT_ref[...]
    qs = lax.dot_general(
        qq.astype(jnp.bfloat16),
        sT.astype(jnp.bfloat16),
        (((1,), (0,)), ((), ())),
        preferred_element_type=jnp.float32,
    )
    q2 = jnp.sum(qq * qq, axis=1, keepdims=True)
    s2 = jnp.sum(sT * sT, axis=0, keepdims=True)
    d2 = (q2 + s2) - 2.0 * qs
    d2 = jnp.maximum(d2, 0.0)
    iota = lax.broadcasted_iota(jnp.int32, (QT, N_S), 1)
    w_cols = []
    w_sum = jnp.zeros((QT, 1), jnp.float32)
    for k in range(K):
        m = jnp.min(d2, axis=1, keepdims=True)
        cand = jnp.where(d2 == m, iota, N_S)
        idx = jnp.min(cand, axis=1, keepdims=True)
        onehot = iota == idx
        w = 1.0 / (jnp.sqrt(m + 1e-12) + EPS)
        w_cols.append(w)
        w_sum = w_sum + w
        d2 = jnp.where(onehot, jnp.inf, d2)
        i_ref[0, :, k : k + 1] = idx + b * N_S
    for k in range(K):
        w_ref[0, :, k : k + 1] = w_cols[k] / w_sum


def _tc_topk(query_coords, s_t):
    B, n_q, _ = query_coords.shape
    return pl.pallas_call(
        _tc_body,
        grid=(B, n_q // QT),
        in_specs=[
            pl.BlockSpec((1, QT, 3), lambda b, t: (b, t, 0)),
            pl.BlockSpec((3, N_S), lambda b, t: (0, 0)),
        ],
        out_specs=(
            pl.BlockSpec((1, QT, K), lambda b, t: (b, t, 0)),
            pl.BlockSpec((1, QT, K), lambda b, t: (b, t, 0)),
        ),
        out_shape=(
            jax.ShapeDtypeStruct((B, n_q, K), jnp.int32),
            jax.ShapeDtypeStruct((B, n_q, K), jnp.float32),
        ),
    )(query_coords, s_t)


def _sc_combine(idx_flat, w_flat, feat_flat, n_total):
    q_per_w = n_total // NW
    n_chunks = q_per_w // QC
    mesh = plsc.VectorSubcoreMesh(core_axis_name="c", subcore_axis_name="s")

    @functools.partial(
        pl.kernel,
        mesh=mesh,
        out_type=jax.ShapeDtypeStruct((n_total, N_F), jnp.float32),
        scratch_types=[
            pltpu.VMEM((2, QC * K), jnp.int32),
            pltpu.VMEM((2, QC * K), jnp.float32),
            pltpu.VMEM((2, QC * K, N_F), jnp.float32),
            pltpu.VMEM((QC, N_F), jnp.float32),
            pltpu.SemaphoreType.DMA,
            pltpu.SemaphoreType.DMA,
        ],
    )
    def body(idx_hbm, w_hbm, feat_hbm, out_hbm, idx_v, w_v, rows_v, out_v, s0, s1):
        wid = lax.axis_index("s") * NC + lax.axis_index("c")
        qbase0 = wid * q_per_w
        sems = (s0, s1)

        def load_and_fire(c, b):
            qbase = qbase0 + c * QC
            pltpu.sync_copy(idx_hbm.at[pl.ds(qbase * K, QC * K)], idx_v.at[b])
            pltpu.sync_copy(w_hbm.at[pl.ds(qbase * K, QC * K)], w_v.at[b])
            pltpu.async_copy(feat_hbm.at[idx_v.at[b]], rows_v.at[b], sems[b])

        load_and_fire(0, 0)
        load_and_fire(1, 1)

        def chunk_pair(p, carry):
            for b in range(2):
                c = p * 2 + b
                qbase = qbase0 + c * QC
                pltpu.make_async_copy(
                    feat_hbm.at[idx_v.at[b]], rows_v.at[b], sems[b]
                ).wait()

                def one_pair(qp, carry2, b=b):
                    wv = w_v[b, pl.ds(qp * 16, 16)]
                    for h in range(2):
                        ql = qp * 2 + h
                        base = ql * K
                        accs = [jnp.zeros((16,), jnp.float32) for _ in range(N_F // 16)]
                        for k in range(K):
                            wk = wv[h * K + k]
                            for j in range(N_F // 16):
                                accs[j] = accs[j] + wk * rows_v[b, base + k, pl.ds(j * 16, 16)]
                        for j in range(N_F // 16):
                            out_v[ql, pl.ds(j * 16, 16)] = accs[j]
                    return carry2

                lax.fori_loop(0, QC // 2, one_pair, 0)
                pltpu.sync_copy(out_v, out_hbm.at[pl.ds(qbase, QC)])

                @pl.when(c + 2 < n_chunks)
                def _fire(b=b, c=c):
                    load_and_fire(c + 2, b)

            return carry

        lax.fori_loop(0, n_chunks // 2, chunk_pair, 0)

    return body(idx_flat, w_flat, feat_flat)


@jax.jit
def kernel(query_coords, sensor_coords, sensor_features):
    B, n_q, _ = query_coords.shape
    s_t = sensor_coords.T
    idx, w = _tc_topk(query_coords, s_t)
    n_total = B * n_q
    out = _sc_combine(
        idx.reshape(n_total * K),
        w.reshape(n_total * K),
        sensor_features.reshape(B * N_S, N_F),
        n_total,
    )
    return out.reshape(B, n_q, N_F)


# speedup vs baseline: 1.2735x; 1.0598x over previous
import functools

import jax
import jax.numpy as jnp
from jax import lax
from jax.experimental import pallas as pl
from jax.experimental.pallas import tpu as pltpu
from jax.experimental.pallas import tpu_sc as plsc

K = 8
EPS = 1e-8
N_S = 2048
N_F = 256
QT = 512

NC = 2
NS = 16
NW = NC * NS
QC = 16


def _tc_body(q_ref, sT_ref, io_ref, i_ref, w_ref):
    b = pl.program_id(0)
    qq = q_ref[0]
    sT = sT_ref[...]
    qs = lax.dot_general(
        qq.astype(jnp.bfloat16),
        sT.astype(jnp.bfloat16),
        (((1,), (0,)), ((), ())),
        preferred_element_type=jnp.float32,
    )
    q2 = jnp.sum(qq * qq, axis=1, keepdims=True)
    s2 = jnp.sum(sT * sT, axis=0, keepdims=True)
    d2 = (q2 + s2) - 2.0 * qs
    d2 = jnp.maximum(d2, 0.0)
    io_cols = io_ref[...]
    w_cols = []
    w_sum = jnp.zeros((QT, 1), jnp.float32)
    for k in range(K):
        m = jnp.min(d2, axis=1, keepdims=True)
        hot = d2 == m
        hotb = jnp.where(hot, 1.0, 0.0).astype(jnp.bfloat16)
        r = lax.dot_general(
            hotb, io_cols, (((1,), (0,)), ((), ())),
            preferred_element_type=jnp.float32,
        )
        idxf = jnp.minimum(r[:, 0:1] + 256.0 * r[:, 1:2], float(N_S - 1))
        w = 1.0 / (jnp.sqrt(m + 1e-12) + EPS)
        w_cols.append(w)
        w_sum = w_sum + w
        d2 = jnp.where(hot, jnp.inf, d2)
        i_ref[0, :, k : k + 1] = idxf.astype(jnp.int32) + b * N_S
    for k in range(K):
        w_ref[0, :, k : k + 1] = w_cols[k] / w_sum


def _tc_topk(query_coords, s_t, io_cols):
    B, n_q, _ = query_coords.shape
    return pl.pallas_call(
        _tc_body,
        grid=(B, n_q // QT),
        in_specs=[
            pl.BlockSpec((1, QT, 3), lambda b, t: (b, t, 0)),
            pl.BlockSpec((3, N_S), lambda b, t: (0, 0)),
            pl.BlockSpec((N_S, 2), lambda b, t: (0, 0)),
        ],
        out_specs=(
            pl.BlockSpec((1, QT, K), lambda b, t: (b, t, 0)),
            pl.BlockSpec((1, QT, K), lambda b, t: (b, t, 0)),
        ),
        out_shape=(
            jax.ShapeDtypeStruct((B, n_q, K), jnp.int32),
            jax.ShapeDtypeStruct((B, n_q, K), jnp.float32),
        ),
    )(query_coords, s_t, io_cols)


def _sc_combine(idx_flat, w_flat, feat_flat, n_total):
    q_per_w = n_total // NW
    n_chunks = q_per_w // QC
    mesh = plsc.VectorSubcoreMesh(core_axis_name="c", subcore_axis_name="s")

    @functools.partial(
        pl.kernel,
        mesh=mesh,
        out_type=jax.ShapeDtypeStruct((n_total, N_F), jnp.float32),
        scratch_types=[
            pltpu.VMEM((2, QC * K), jnp.int32),
            pltpu.VMEM((2, QC * K), jnp.float32),
            pltpu.VMEM((2, QC * K, N_F), jnp.float32),
            pltpu.VMEM((QC, N_F), jnp.float32),
            pltpu.SemaphoreType.DMA,
            pltpu.SemaphoreType.DMA,
        ],
    )
    def body(idx_hbm, w_hbm, feat_hbm, out_hbm, idx_v, w_v, rows_v, out_v, s0, s1):
        wid = lax.axis_index("s") * NC + lax.axis_index("c")
        qbase0 = wid * q_per_w
        sems = (s0, s1)

        def load_and_fire(c, b):
            qbase = qbase0 + c * QC
            pltpu.sync_copy(idx_hbm.at[pl.ds(qbase * K, QC * K)], idx_v.at[b])
            pltpu.sync_copy(w_hbm.at[pl.ds(qbase * K, QC * K)], w_v.at[b])
            pltpu.async_copy(feat_hbm.at[idx_v.at[b]], rows_v.at[b], sems[b])

        load_and_fire(0, 0)
        load_and_fire(1, 1)

        def chunk_pair(p, carry):
            for b in range(2):
                c = p * 2 + b
                qbase = qbase0 + c * QC
                pltpu.make_async_copy(
                    feat_hbm.at[idx_v.at[b]], rows_v.at[b], sems[b]
                ).wait()

                def one_pair(qp, carry2, b=b):
                    wv = w_v[b, pl.ds(qp * 16, 16)]
                    for h in range(2):
                        ql = qp * 2 + h
                        base = ql * K
                        accs = [jnp.zeros((16,), jnp.float32) for _ in range(N_F // 16)]
                        for k in range(K):
                            wk = wv[h * K + k]
                            for j in range(N_F // 16):
                                accs[j] = accs[j] + wk * rows_v[b, base + k, pl.ds(j * 16, 16)]
                        for j in range(N_F // 16):
                            out_v[ql, pl.ds(j * 16, 16)] = accs[j]
                    return carry2

                lax.fori_loop(0, QC // 2, one_pair, 0)
                pltpu.sync_copy(out_v, out_hbm.at[pl.ds(qbase, QC)])

                @pl.when(c + 2 < n_chunks)
                def _fire(b=b, c=c):
                    load_and_fire(c + 2, b)

            return carry

        lax.fori_loop(0, n_chunks // 2, chunk_pair, 0)

    return body(idx_flat, w_flat, feat_flat)


@jax.jit
def kernel(query_coords, sensor_coords, sensor_features):
    B, n_q, _ = query_coords.shape
    s_t = sensor_coords.T
    iota = jnp.arange(N_S, dtype=jnp.int32)
    io_cols = jnp.stack([iota % 256, iota // 256], axis=1).astype(jnp.bfloat16)
    idx, w = _tc_topk(query_coords, s_t, io_cols)
    n_total = B * n_q
    out = _sc_combine(
        idx.reshape(n_total * K),
        w.reshape(n_total * K),
        sensor_features.reshape(B * N_S, N_F),
        n_total,
    )
    return out.reshape(B, n_q, N_F)


# speedup vs baseline: 1.3759x; 1.0804x over previous
import functools

import jax
import jax.numpy as jnp
from jax import lax
from jax.experimental import pallas as pl
from jax.experimental.pallas import tpu as pltpu
from jax.experimental.pallas import tpu_sc as plsc

K = 8
EPS = 1e-8
N_S = 2048
N_F = 256
QT = 512

NC = 2
NS = 16
NW = NC * NS
QC = 16


def _tc_body(q_ref, sT_ref, i_ref, w_ref):
    b = pl.program_id(0)
    qq = q_ref[0]
    sT = sT_ref[...]
    qs = lax.dot_general(
        qq.astype(jnp.bfloat16),
        sT.astype(jnp.bfloat16),
        (((1,), (0,)), ((), ())),
        preferred_element_type=jnp.float32,
    )
    q2 = jnp.sum(qq * qq, axis=1, keepdims=True)
    s2 = jnp.sum(sT * sT, axis=0, keepdims=True)
    d2 = (q2 + s2) - 2.0 * qs
    d2 = jnp.maximum(d2, 0.0)
    iotaf = lax.broadcasted_iota(jnp.int32, (QT, N_S), 1).astype(jnp.float32)
    w_cols = []
    w_sum = jnp.zeros((QT, 1), jnp.float32)
    for k in range(K):
        m = jnp.min(d2, axis=1, keepdims=True)
        cand = jnp.where(d2 == m, iotaf, float(N_S))
        idxf = jnp.min(cand, axis=1, keepdims=True)
        onehot = iotaf == idxf
        w = 1.0 / (jnp.sqrt(m + 1e-12) + EPS)
        w_cols.append(w)
        w_sum = w_sum + w
        d2 = jnp.where(onehot, jnp.inf, d2)
        i_ref[0, :, k : k + 1] = idxf.astype(jnp.int32) + b * N_S
    for k in range(K):
        w_ref[0, :, k : k + 1] = w_cols[k] / w_sum


def _tc_topk(query_coords, s_t):
    B, n_q, _ = query_coords.shape
    return pl.pallas_call(
        _tc_body,
        grid=(B, n_q // QT),
        in_specs=[
            pl.BlockSpec((1, QT, 3), lambda b, t: (b, t, 0)),
            pl.BlockSpec((3, N_S), lambda b, t: (0, 0)),
        ],
        out_specs=(
            pl.BlockSpec((1, QT, K), lambda b, t: (b, t, 0)),
            pl.BlockSpec((1, QT, K), lambda b, t: (b, t, 0)),
        ),
        out_shape=(
            jax.ShapeDtypeStruct((B, n_q, K), jnp.int32),
            jax.ShapeDtypeStruct((B, n_q, K), jnp.float32),
        ),
    )(query_coords, s_t)


def _sc_combine(idx_flat, w_flat, feat_flat, n_total):
    q_per_w = n_total // NW
    n_chunks = q_per_w // QC
    mesh = plsc.VectorSubcoreMesh(core_axis_name="c", subcore_axis_name="s")

    @functools.partial(
        pl.kernel,
        mesh=mesh,
        out_type=jax.ShapeDtypeStruct((n_total, N_F), jnp.float32),
        scratch_types=[
            pltpu.VMEM((2, QC * K), jnp.int32),
            pltpu.VMEM((2, QC * K), jnp.float32),
            pltpu.VMEM((2, QC * K, N_F), jnp.float32),
            pltpu.VMEM((QC, N_F), jnp.float32),
            pltpu.SemaphoreType.DMA,
            pltpu.SemaphoreType.DMA,
        ],
    )
    def body(idx_hbm, w_hbm, feat_hbm, out_hbm, idx_v, w_v, rows_v, out_v, s0, s1):
        wid = lax.axis_index("s") * NC + lax.axis_index("c")
        qbase0 = wid * q_per_w
        sems = (s0, s1)

        def load_and_fire(c, b):
            qbase = qbase0 + c * QC
            pltpu.sync_copy(idx_hbm.at[pl.ds(qbase * K, QC * K)], idx_v.at[b])
            pltpu.sync_copy(w_hbm.at[pl.ds(qbase * K, QC * K)], w_v.at[b])
            pltpu.async_copy(feat_hbm.at[idx_v.at[b]], rows_v.at[b], sems[b])

        load_and_fire(0, 0)
        load_and_fire(1, 1)

        def chunk_pair(p, carry):
            for b in range(2):
                c = p * 2 + b
                qbase = qbase0 + c * QC
                pltpu.make_async_copy(
                    feat_hbm.at[idx_v.at[b]], rows_v.at[b], sems[b]
                ).wait()

                def one_pair(qp, carry2, b=b):
                    wv = w_v[b, pl.ds(qp * 16, 16)]
                    for h in range(2):
                        ql = qp * 2 + h
                        base = ql * K
                        accs = [jnp.zeros((16,), jnp.float32) for _ in range(N_F // 16)]
                        for k in range(K):
                            wk = wv[h * K + k]
                            for j in range(N_F // 16):
                                accs[j] = accs[j] + wk * rows_v[b, base + k, pl.ds(j * 16, 16)]
                        for j in range(N_F // 16):
                            out_v[ql, pl.ds(j * 16, 16)] = accs[j]
                    return carry2

                lax.fori_loop(0, QC // 2, one_pair, 0)
                pltpu.sync_copy(out_v, out_hbm.at[pl.ds(qbase, QC)])

                @pl.when(c + 2 < n_chunks)
                def _fire(b=b, c=c):
                    load_and_fire(c + 2, b)

            return carry

        lax.fori_loop(0, n_chunks // 2, chunk_pair, 0)

    return body(idx_flat, w_flat, feat_flat)


@jax.jit
def kernel(query_coords, sensor_coords, sensor_features):
    B, n_q, _ = query_coords.shape
    s_t = sensor_coords.T
    idx, w = _tc_topk(query_coords, s_t)
    n_total = B * n_q
    out = _sc_combine(
        idx.reshape(n_total * K),
        w.reshape(n_total * K),
        sensor_features.reshape(B * N_S, N_F),
        n_total,
    )
    return out.reshape(B, n_q, N_F)
